# Initial kernel scaffold; baseline (speedup 1.0000x reference)
import jax
import jax.numpy as jnp
from jax.experimental import pallas as pl


def kernel(h, edge_index, Wd, bd, Wf, bf, W, b, Wa, ba):
    raise NotImplementedError("write your pallas kernel here")


# speedup vs baseline: 40.2887x; 40.2887x over previous
import functools

import jax
import jax.numpy as jnp
from jax import lax
from jax.experimental import pallas as pl
from jax.experimental.pallas import tpu as pltpu
from jax.experimental.pallas import tpu_sc as plsc

N = 10000
E = 320000
D = 128
HEADS = 4
HF = 32
DR = 32

NC = 2
NS = 16
NW = NC * NS
ROW = 144
SROW = 176
DROW = 48
HS0 = D + 1 + HEADS
HD0 = 1 + HEADS
EPW = E // NW
CH = 80
NCHUNK = EPW // CH
NPW = N // NS
ZR = 25

RB = 2000


def _pre_body(h_ref, W_ref, b_ref, Wd_ref, bd_ref, wf1_ref, wf2_ref,
              P_ref, Q_ref, bf_ref, ba_ref, src_ref, dst_ref):
    bf16 = jnp.bfloat16
    h = h_ref[...]
    hb = h.astype(bf16)
    h2 = jnp.dot(hb, W_ref[...].astype(bf16),
                 preferred_element_type=jnp.float32) + b_ref[...]
    hdw = jnp.dot(hb, Wd_ref[...].astype(bf16),
                  preferred_element_type=jnp.float32) + bd_ref[...]
    hdwb = hdw.astype(bf16)
    a1 = jnp.dot(hdwb, wf1_ref[...].astype(bf16),
                 preferred_element_type=jnp.float32) + bf_ref[...]
    a2 = jnp.dot(hdwb, wf2_ref[...].astype(bf16),
                 preferred_element_type=jnp.float32)
    h2b = h2.astype(bf16)
    p = jnp.dot(h2b, P_ref[...].astype(bf16),
                preferred_element_type=jnp.float32)
    q = jnp.dot(h2b, Q_ref[...].astype(bf16),
                preferred_element_type=jnp.float32) + ba_ref[...]
    zs = jnp.zeros((h.shape[0], SROW - D - 1 - HEADS - DR), jnp.float32)
    zd = jnp.zeros((h.shape[0], DROW - 1 - HEADS - DR), jnp.float32)
    src_ref[...] = jnp.concatenate([h2, a1, p, hdw, zs], axis=1)
    dst_ref[...] = jnp.concatenate([a2, q, hdw, zd], axis=1)


def _fin_body(acc_ref, R_ref, o_ref):
    s = acc_ref[0] + acc_ref[1]
    d = s[:, D:D + HEADS]
    r = 1.0 / jnp.maximum(d, 1e-9)
    rexp = jnp.dot(r, R_ref[...], preferred_element_type=jnp.float32)
    o_ref[...] = s[:, :D] * rexp


def _bf16r(x):
    u = plsc.bitcast(x, jnp.uint32)
    r = (u + jnp.uint32(0x7FFF) + ((u >> jnp.uint32(16)) & jnp.uint32(1))) \
        & jnp.uint32(0xFFFF0000)
    return plsc.bitcast(r, jnp.float32)


def _edge_body(edge_hbm, srcdata_hbm, dstdata_hbm, wf3_hbm, out_hbm,
               srcidx, dstidx, srcbuf, dstbuf, scaled, cbuf, aebuf, zbuf,
               wf3buf, acc, sem1, sem2):
    cid = lax.axis_index("c")
    sid = lax.axis_index("s")
    wid = cid * NS + sid
    ebase = wid * EPW
    lanes = jnp.arange(NS, dtype=jnp.int32)
    pltpu.sync_copy(wf3_hbm, wf3buf.at[pl.ds(16, DR)])

    def zrow(r, _):
        def zcol(cc, __):
            zbuf[r, pl.ds(cc * 16, 16)] = jnp.zeros((16,), jnp.float32)
            return 0
        lax.fori_loop(0, ROW // 16, zcol, 0)
        return 0
    lax.fori_loop(0, ZR, zrow, 0)
    for k in range(NPW // ZR):
        pltpu.sync_copy(zbuf, acc.at[pl.ds(sid * NPW + k * ZR, ZR)])
    plsc.subcore_barrier()

    def chunk(c, _):
        base = ebase + c * CH
        d1 = pltpu.async_copy(edge_hbm.at[0, pl.ds(base, CH)], srcidx, sem1)
        d2 = pltpu.async_copy(edge_hbm.at[1, pl.ds(base, CH)], dstidx, sem2)
        d1.wait()
        d2.wait()
        g1 = pltpu.async_copy(srcdata_hbm.at[srcidx], srcbuf, sem1)
        g2 = pltpu.async_copy(dstdata_hbm.at[dstidx], dstbuf, sem2)
        g1.wait()
        g2.wait()
        for g in range(CH // 16):
            rows = lanes + g * 16
            a16 = plsc.load_gather(srcbuf, [rows, jnp.full((16,), D, jnp.int32)])
            b16 = plsc.load_gather(dstbuf, [rows, jnp.full((16,), 0, jnp.int32)])
            t3 = jnp.zeros((16,), jnp.float32)
            for k in range(DR):
                hsk = plsc.load_gather(
                    srcbuf, [rows, jnp.full((16,), HS0 + k, jnp.int32)])
                hdk = plsc.load_gather(
                    dstbuf, [rows, jnp.full((16,), HD0 + k, jnp.int32)])
                wk = plsc.load_gather(
                    wf3buf, [jnp.full((16,), 16 + k, jnp.int32)])
                t3 = t3 + _bf16r(hsk - hdk) * wk
            s16 = jnp.sign(a16 + b16 + t3)
            for hh in range(HEADS):
                p16 = plsc.load_gather(
                    srcbuf, [rows, jnp.full((16,), D + 1 + hh, jnp.int32)])
                q16 = plsc.load_gather(
                    dstbuf, [rows, jnp.full((16,), 1 + hh, jnp.int32)])
                al = s16 * p16 + q16
                al = jnp.where(al >= 0.0, al, 0.01 * al)
                ae = jnp.exp(al)
                cbuf[pl.ds((hh + 1) * 16, 16)] = ae * s16
                aebuf[pl.ds(hh * 16, 16)] = ae
            for il in range(16):
                i = g * 16 + il
                ws = [plsc.load_gather(
                          cbuf, [jnp.full((16,), (hh + 1) * 16 + il, jnp.int32)])
                      for hh in range(HEADS)]
                for j in range(D // 16):
                    scaled[i, pl.ds(j * 16, 16)] = (
                        srcbuf[i, pl.ds(j * 16, 16)] * ws[j * 16 // HF])
                tmask = lanes < HEADS
                tidx = jnp.where(tmask, lanes * 16 + il, 0)
                tv = plsc.load_gather(aebuf, [tidx])
                scaled[i, pl.ds(D, 16)] = jnp.where(tmask, tv, 0.0)
        pltpu.sync_copy(scaled, acc.at[dstidx], add=True)
        return 0

    lax.fori_loop(0, NCHUNK, chunk, 0)
    plsc.subcore_barrier()

    pltpu.sync_copy(acc.at[pl.ds(sid * NPW, NPW)],
                    out_hbm.at[cid, pl.ds(sid * NPW, NPW)])


_edge_call_cache = []


def _edge_call(*args):
    if not _edge_call_cache:
        _edge_call_cache.append(_build_edge_call())
    return _edge_call_cache[0](*args)


def _build_edge_call():
    return functools.partial(
        pl.kernel,
        out_type=jax.ShapeDtypeStruct((NC, N, ROW), jnp.float32),
        mesh=plsc.VectorSubcoreMesh(core_axis_name="c", subcore_axis_name="s",
                                    num_cores=NC, num_subcores=NS),
        compiler_params=pltpu.CompilerParams(use_tc_tiling_on_sc=False,
                                             needs_layout_passes=False),
        scratch_types=[
            pltpu.VMEM((CH,), jnp.int32),
            pltpu.VMEM((CH,), jnp.int32),
            pltpu.VMEM((CH, SROW), jnp.float32),
            pltpu.VMEM((CH, DROW), jnp.float32),
            pltpu.VMEM((CH, ROW), jnp.float32),
            pltpu.VMEM((80,), jnp.float32),
            pltpu.VMEM((64,), jnp.float32),
            pltpu.VMEM((ZR, ROW), jnp.float32),
            pltpu.VMEM((16 + DR,), jnp.float32),
            pltpu.VMEM_SHARED((N, ROW), jnp.float32),
            pltpu.SemaphoreType.DMA,
            pltpu.SemaphoreType.DMA,
        ],
    )(_edge_body)


def kernel(h, edge_index, Wd, bd, Wf, bf, W, b, Wa, ba):
    wf1 = Wf[:DR]
    wf2 = Wf[DR:2 * DR]
    wf3r = Wf[2 * DR:, 0].astype(jnp.bfloat16).astype(jnp.float32)
    hmask = (jnp.arange(D)[:, None] // HF) == jnp.arange(HEADS)[None, :]
    P = jnp.where(hmask, jnp.tile(Wa[:HF, 0], HEADS)[:, None], 0.0)
    Q = jnp.where(hmask, jnp.tile(Wa[HF:, 0], HEADS)[:, None], 0.0)
    R = hmask.T.astype(jnp.float32)

    srcdata, dstdata = pl.pallas_call(
        _pre_body,
        grid=(N // RB,),
        in_specs=[
            pl.BlockSpec((RB, D), lambda i: (i, 0)),
            pl.BlockSpec((D, D), lambda i: (0, 0)),
            pl.BlockSpec((1, D), lambda i: (0, 0)),
            pl.BlockSpec((D, DR), lambda i: (0, 0)),
            pl.BlockSpec((1, DR), lambda i: (0, 0)),
            pl.BlockSpec((DR, 1), lambda i: (0, 0)),
            pl.BlockSpec((DR, 1), lambda i: (0, 0)),
            pl.BlockSpec((D, HEADS), lambda i: (0, 0)),
            pl.BlockSpec((D, HEADS), lambda i: (0, 0)),
            pl.BlockSpec((1, 1), lambda i: (0, 0)),
            pl.BlockSpec((1, 1), lambda i: (0, 0)),
        ],
        out_specs=[
            pl.BlockSpec((RB, SROW), lambda i: (i, 0)),
            pl.BlockSpec((RB, DROW), lambda i: (i, 0)),
        ],
        out_shape=[
            jax.ShapeDtypeStruct((N, SROW), jnp.float32),
            jax.ShapeDtypeStruct((N, DROW), jnp.float32),
        ],
    )(h, W, b.reshape(1, D), Wd, bd.reshape(1, DR), wf1, wf2, P, Q,
      bf.reshape(1, 1), ba.reshape(1, 1))

    acc = _edge_call(edge_index, srcdata, dstdata, wf3r)

    out = pl.pallas_call(
        _fin_body,
        grid=(N // RB,),
        in_specs=[
            pl.BlockSpec((NC, RB, ROW), lambda i: (0, i, 0)),
            pl.BlockSpec((HEADS, D), lambda i: (0, 0)),
        ],
        out_specs=pl.BlockSpec((RB, D), lambda i: (i, 0)),
        out_shape=jax.ShapeDtypeStruct((N, D), jnp.float32),
    )(acc, R)
    return out
